# initial kernel scaffold (unmeasured)
import jax
import jax.numpy as jnp
from jax import lax
from jax.experimental import pallas as pl
from jax.experimental.pallas import tpu as pltpu

N_DEV = 4


def kernel(x, w_mat):
    m_per, k = x.shape
    _, n_per = w_mat.shape
    M = N_DEV * m_per
    half = m_per // 2

    def gelu(y):
        c = 0.7978845608028654
        return 0.5 * y * (1.0 + jnp.tanh(c * (y + 0.044715 * y * y * y)))

    def body(x_ref, w_ref, out_ref, xg_ref, send_sems, recv_sems):
        my = lax.axis_index("i")
        left = lax.rem(my + N_DEV - 1, N_DEV)
        right = lax.rem(my + 1, N_DEV)
        opp = lax.rem(my + 2, N_DEV)

        barrier_sem = pltpu.get_barrier_semaphore()
        for nbr in (left, right):
            pl.semaphore_signal(
                barrier_sem, inc=1,
                device_id=(nbr,), device_id_type=pl.DeviceIdType.MESH,
            )
        pl.semaphore_wait(barrier_sem, 2)

        def rows(o):
            return pl.ds(o * m_per, m_per)

        def half_rows(o, h):
            return pl.ds(o * m_per + h * half, half)

        def mm(origin_rows):
            acc = jnp.dot(
                xg_ref[origin_rows, :], w_ref[...],
                preferred_element_type=jnp.float32,
            )
            return gelu(acc)

        send_own_r = pltpu.make_async_remote_copy(
            src_ref=x_ref, dst_ref=xg_ref.at[rows(my)],
            send_sem=send_sems.at[0], recv_sem=recv_sems.at[0],
            device_id=(right,), device_id_type=pl.DeviceIdType.MESH,
        )
        send_own_r.start()
        send_own_l = pltpu.make_async_remote_copy(
            src_ref=x_ref, dst_ref=xg_ref.at[rows(my)],
            send_sem=send_sems.at[1], recv_sem=recv_sems.at[1],
            device_id=(left,), device_id_type=pl.DeviceIdType.MESH,
        )
        send_own_l.start()

        out_ref[rows(my), :] = gelu(jnp.dot(
            x_ref[...], w_ref[...], preferred_element_type=jnp.float32,
        ))

        recv_from_l = pltpu.make_async_remote_copy(
            src_ref=xg_ref.at[rows(left)], dst_ref=xg_ref.at[rows(left)],
            send_sem=send_sems.at[0], recv_sem=recv_sems.at[0],
            device_id=(left,), device_id_type=pl.DeviceIdType.MESH,
        )
        recv_from_l.wait_recv()
        fwd_r = pltpu.make_async_remote_copy(
            src_ref=xg_ref.at[half_rows(left, 0)],
            dst_ref=xg_ref.at[half_rows(left, 0)],
            send_sem=send_sems.at[2], recv_sem=recv_sems.at[2],
            device_id=(right,), device_id_type=pl.DeviceIdType.MESH,
        )
        fwd_r.start()

        recv_from_r = pltpu.make_async_remote_copy(
            src_ref=xg_ref.at[rows(right)], dst_ref=xg_ref.at[rows(right)],
            send_sem=send_sems.at[1], recv_sem=recv_sems.at[1],
            device_id=(right,), device_id_type=pl.DeviceIdType.MESH,
        )
        recv_from_r.wait_recv()
        fwd_l = pltpu.make_async_remote_copy(
            src_ref=xg_ref.at[half_rows(right, 1)],
            dst_ref=xg_ref.at[half_rows(right, 1)],
            send_sem=send_sems.at[3], recv_sem=recv_sems.at[3],
            device_id=(left,), device_id_type=pl.DeviceIdType.MESH,
        )
        fwd_l.start()

        out_ref[rows(left), :] = mm(rows(left))
        out_ref[rows(right), :] = mm(rows(right))

        recv_h0 = pltpu.make_async_remote_copy(
            src_ref=xg_ref.at[half_rows(opp, 0)],
            dst_ref=xg_ref.at[half_rows(opp, 0)],
            send_sem=send_sems.at[2], recv_sem=recv_sems.at[2],
            device_id=(left,), device_id_type=pl.DeviceIdType.MESH,
        )
        recv_h0.wait_recv()
        recv_h1 = pltpu.make_async_remote_copy(
            src_ref=xg_ref.at[half_rows(opp, 1)],
            dst_ref=xg_ref.at[half_rows(opp, 1)],
            send_sem=send_sems.at[3], recv_sem=recv_sems.at[3],
            device_id=(right,), device_id_type=pl.DeviceIdType.MESH,
        )
        recv_h1.wait_recv()

        out_ref[rows(opp), :] = mm(rows(opp))

        send_own_r.wait_send()
        send_own_l.wait_send()
        fwd_r.wait_send()
        fwd_l.wait_send()

    return pl.pallas_call(
        body,
        out_shape=jax.ShapeDtypeStruct((M, n_per), jnp.float32),
        in_specs=[
            pl.BlockSpec(memory_space=pltpu.VMEM),
            pl.BlockSpec(memory_space=pltpu.VMEM),
        ],
        out_specs=pl.BlockSpec(memory_space=pltpu.VMEM),
        scratch_shapes=[
            pltpu.VMEM((M, k), x.dtype),
            pltpu.SemaphoreType.DMA((4,)),
            pltpu.SemaphoreType.DMA((4,)),
        ],
        compiler_params=pltpu.CompilerParams(collective_id=0),
    )(x, w_mat)


# baseline (device time: 175651 ns/iter reference)
import jax
import jax.numpy as jnp
from jax import lax
from jax.experimental import pallas as pl
from jax.experimental.pallas import tpu as pltpu

N_DEV = 4


def kernel(x, w_mat):
    x = x.astype(jnp.bfloat16)
    w_mat = w_mat.astype(jnp.bfloat16)
    m_per, k = x.shape
    _, n_per = w_mat.shape
    M = N_DEV * m_per
    half = m_per // 2

    def gelu(y):
        c = 0.7978845608028654
        return 0.5 * y * (1.0 + jnp.tanh(c * (y + 0.044715 * y * y * y)))

    def body(x_ref, w_ref, out_ref, xg_ref, send_sems, recv_sems):
        my = lax.axis_index("i")
        left = lax.rem(my + N_DEV - 1, N_DEV)
        right = lax.rem(my + 1, N_DEV)
        opp = lax.rem(my + 2, N_DEV)

        barrier_sem = pltpu.get_barrier_semaphore()
        for nbr in (left, right):
            pl.semaphore_signal(
                barrier_sem, inc=1,
                device_id=(nbr,), device_id_type=pl.DeviceIdType.MESH,
            )
        pl.semaphore_wait(barrier_sem, 2)

        def rows(o):
            return pl.ds(o * m_per, m_per)

        def half_rows(o, h):
            return pl.ds(o * m_per + h * half, half)

        def mm(origin_rows):
            acc = jnp.dot(
                xg_ref[origin_rows, :], w_ref[...],
                preferred_element_type=jnp.float32,
            )
            return gelu(acc)

        send_own_r = pltpu.make_async_remote_copy(
            src_ref=x_ref, dst_ref=xg_ref.at[rows(my)],
            send_sem=send_sems.at[0], recv_sem=recv_sems.at[0],
            device_id=(right,), device_id_type=pl.DeviceIdType.MESH,
        )
        send_own_r.start()
        send_own_l = pltpu.make_async_remote_copy(
            src_ref=x_ref, dst_ref=xg_ref.at[rows(my)],
            send_sem=send_sems.at[1], recv_sem=recv_sems.at[1],
            device_id=(left,), device_id_type=pl.DeviceIdType.MESH,
        )
        send_own_l.start()

        out_ref[rows(my), :] = gelu(jnp.dot(
            x_ref[...], w_ref[...], preferred_element_type=jnp.float32,
        ))

        recv_from_l = pltpu.make_async_remote_copy(
            src_ref=xg_ref.at[rows(left)], dst_ref=xg_ref.at[rows(left)],
            send_sem=send_sems.at[0], recv_sem=recv_sems.at[0],
            device_id=(left,), device_id_type=pl.DeviceIdType.MESH,
        )
        recv_from_l.wait_recv()
        fwd_r = pltpu.make_async_remote_copy(
            src_ref=xg_ref.at[half_rows(left, 0)],
            dst_ref=xg_ref.at[half_rows(left, 0)],
            send_sem=send_sems.at[2], recv_sem=recv_sems.at[2],
            device_id=(right,), device_id_type=pl.DeviceIdType.MESH,
        )
        fwd_r.start()

        recv_from_r = pltpu.make_async_remote_copy(
            src_ref=xg_ref.at[rows(right)], dst_ref=xg_ref.at[rows(right)],
            send_sem=send_sems.at[1], recv_sem=recv_sems.at[1],
            device_id=(right,), device_id_type=pl.DeviceIdType.MESH,
        )
        recv_from_r.wait_recv()
        fwd_l = pltpu.make_async_remote_copy(
            src_ref=xg_ref.at[half_rows(right, 1)],
            dst_ref=xg_ref.at[half_rows(right, 1)],
            send_sem=send_sems.at[3], recv_sem=recv_sems.at[3],
            device_id=(left,), device_id_type=pl.DeviceIdType.MESH,
        )
        fwd_l.start()

        out_ref[rows(left), :] = mm(rows(left))
        out_ref[rows(right), :] = mm(rows(right))

        recv_h0 = pltpu.make_async_remote_copy(
            src_ref=xg_ref.at[half_rows(opp, 0)],
            dst_ref=xg_ref.at[half_rows(opp, 0)],
            send_sem=send_sems.at[2], recv_sem=recv_sems.at[2],
            device_id=(left,), device_id_type=pl.DeviceIdType.MESH,
        )
        recv_h0.wait_recv()
        recv_h1 = pltpu.make_async_remote_copy(
            src_ref=xg_ref.at[half_rows(opp, 1)],
            dst_ref=xg_ref.at[half_rows(opp, 1)],
            send_sem=send_sems.at[3], recv_sem=recv_sems.at[3],
            device_id=(right,), device_id_type=pl.DeviceIdType.MESH,
        )
        recv_h1.wait_recv()

        out_ref[rows(opp), :] = mm(rows(opp))

        send_own_r.wait_send()
        send_own_l.wait_send()
        fwd_r.wait_send()
        fwd_l.wait_send()

    return pl.pallas_call(
        body,
        out_shape=jax.ShapeDtypeStruct((M, n_per), jnp.float32),
        in_specs=[
            pl.BlockSpec(memory_space=pltpu.VMEM),
            pl.BlockSpec(memory_space=pltpu.VMEM),
        ],
        out_specs=pl.BlockSpec(memory_space=pltpu.VMEM),
        scratch_shapes=[
            pltpu.VMEM((M, k), x.dtype),
            pltpu.SemaphoreType.DMA((4,)),
            pltpu.SemaphoreType.DMA((4,)),
        ],
        compiler_params=pltpu.CompilerParams(
            collective_id=0,
            vmem_limit_bytes=63 * 1024 * 1024,
        ),
    )(x, w_mat)


# device time: 173044 ns/iter; 1.0151x vs baseline; 1.0151x over previous
import jax
import jax.numpy as jnp
from jax import lax
from jax.experimental import pallas as pl
from jax.experimental.pallas import tpu as pltpu

N_DEV = 4


def kernel(x, w_mat):
    x = x.astype(jnp.bfloat16)
    w_mat = w_mat.astype(jnp.bfloat16)
    m_per, k = x.shape
    _, n_per = w_mat.shape
    M = N_DEV * m_per
    half = m_per // 2
    quart = m_per // 4

    def gelu(y):
        c = 0.7978845608028654
        return 0.5 * y * (1.0 + jnp.tanh(c * (y + 0.044715 * y * y * y)))

    def body(x_ref, w_ref, out_ref, xg_ref, send_sems, recv_sems):
        my = lax.axis_index("i")
        left = lax.rem(my + N_DEV - 1, N_DEV)
        right = lax.rem(my + 1, N_DEV)
        opp = lax.rem(my + 2, N_DEV)

        barrier_sem = pltpu.get_barrier_semaphore()
        for nbr in (left, right):
            pl.semaphore_signal(
                barrier_sem, inc=1,
                device_id=(nbr,), device_id_type=pl.DeviceIdType.MESH,
            )
        pl.semaphore_wait(barrier_sem, 2)

        def rows(o, start, size):
            return pl.ds(o * m_per + start, size)

        def copy(si, ri, origin, start, size, target):
            return pltpu.make_async_remote_copy(
                src_ref=xg_ref.at[rows(origin, start, size)],
                dst_ref=xg_ref.at[rows(origin, start, size)],
                send_sem=send_sems.at[si], recv_sem=recv_sems.at[ri],
                device_id=(target,), device_id_type=pl.DeviceIdType.MESH,
            )

        def mm(origin, start, size):
            acc = jnp.dot(
                xg_ref[rows(origin, start, size), :], w_ref[...],
                preferred_element_type=jnp.float32,
            )
            out_ref[rows(origin, start, size), :] = gelu(acc)

        xg_ref[rows(my, 0, m_per), :] = x_ref[...]

        s_r_h0 = copy(0, 0, my, 0, half, right)
        s_r_h0.start()
        s_l_h1 = copy(4, 4, my, half, half, left)
        s_l_h1.start()
        s_r_h1 = copy(1, 1, my, half, half, right)
        s_r_h1.start()
        s_l_h0 = copy(5, 5, my, 0, half, left)
        s_l_h0.start()

        mm(my, 0, m_per)

        copy(0, 0, left, 0, half, left).wait_recv()
        f_r_q0 = copy(2, 2, left, 0, quart, right)
        f_r_q0.start()
        f_r_q1 = copy(3, 3, left, quart, quart, right)
        f_r_q1.start()

        copy(4, 4, right, half, half, right).wait_recv()
        f_l_q0 = copy(6, 6, right, half, quart, left)
        f_l_q0.start()
        f_l_q1 = copy(7, 7, right, half + quart, quart, left)
        f_l_q1.start()

        mm(left, 0, half)
        mm(right, half, half)

        copy(1, 1, left, half, half, left).wait_recv()
        mm(left, half, half)
        copy(5, 5, right, 0, half, right).wait_recv()
        mm(right, 0, half)

        copy(2, 2, opp, 0, quart, left).wait_recv()
        mm(opp, 0, quart)
        copy(6, 6, opp, half, quart, right).wait_recv()
        mm(opp, half, quart)
        copy(3, 3, opp, quart, quart, left).wait_recv()
        mm(opp, quart, quart)
        copy(7, 7, opp, half + quart, quart, right).wait_recv()
        mm(opp, half + quart, quart)

        s_r_h0.wait_send()
        s_l_h1.wait_send()
        s_r_h1.wait_send()
        s_l_h0.wait_send()
        f_r_q0.wait_send()
        f_r_q1.wait_send()
        f_l_q0.wait_send()
        f_l_q1.wait_send()

    return pl.pallas_call(
        body,
        out_shape=jax.ShapeDtypeStruct((M, n_per), jnp.float32),
        in_specs=[
            pl.BlockSpec(memory_space=pltpu.VMEM),
            pl.BlockSpec(memory_space=pltpu.VMEM),
        ],
        out_specs=pl.BlockSpec(memory_space=pltpu.VMEM),
        scratch_shapes=[
            pltpu.VMEM((M, k), jnp.bfloat16),
            pltpu.SemaphoreType.DMA((8,)),
            pltpu.SemaphoreType.DMA((8,)),
        ],
        compiler_params=pltpu.CompilerParams(
            collective_id=0,
            vmem_limit_bytes=63 * 1024 * 1024,
        ),
    )(x, w_mat)


# device time: 164838 ns/iter; 1.0656x vs baseline; 1.0498x over previous
import jax
import jax.numpy as jnp
from jax import lax
from jax.experimental import pallas as pl
from jax.experimental.pallas import tpu as pltpu

N_DEV = 4


def kernel(x, w_mat):
    w_mat = w_mat.astype(jnp.bfloat16)
    m_per, k = x.shape
    _, n_per = w_mat.shape
    M = N_DEV * m_per
    quart = m_per // 4
    blk = 128

    def gelu(y):
        c = 0.7978845608028654
        return 0.5 * y * (1.0 + jnp.tanh(c * (y + 0.044715 * y * y * y)))

    def body(x_ref, w_ref, out_ref, xg_ref, send_sems, recv_sems):
        my = lax.axis_index("i")
        left = lax.rem(my + N_DEV - 1, N_DEV)
        right = lax.rem(my + 1, N_DEV)
        opp = lax.rem(my + 2, N_DEV)

        barrier_sem = pltpu.get_barrier_semaphore()
        for nbr in (left, right):
            pl.semaphore_signal(
                barrier_sem, inc=1,
                device_id=(nbr,), device_id_type=pl.DeviceIdType.MESH,
            )
        pl.semaphore_wait(barrier_sem, 2)

        def rows(o, start, size):
            return pl.ds(o * m_per + start, size)

        def copy(si, ri, origin, q, target):
            return pltpu.make_async_remote_copy(
                src_ref=xg_ref.at[rows(origin, q * quart, quart)],
                dst_ref=xg_ref.at[rows(origin, q * quart, quart)],
                send_sem=send_sems.at[si], recv_sem=recv_sems.at[ri],
                device_id=(target,), device_id_type=pl.DeviceIdType.MESH,
            )

        def mm(origin, start, size):
            acc = jnp.dot(
                xg_ref[rows(origin, start, size), :], w_ref[...],
                preferred_element_type=jnp.float32,
            )
            out_ref[rows(origin, start, size), :] = gelu(acc)

        def conv(q):
            for b in range(quart // blk):
                r = q * quart + b * blk
                xg_ref[rows(my, r, blk), :] = (
                    x_ref[pl.ds(r, blk), :].astype(jnp.bfloat16)
                )

        conv(0)
        own_r_q0 = copy(0, 0, my, 0, right)
        own_r_q0.start()
        conv(2)
        own_l_q2 = copy(6, 6, my, 2, left)
        own_l_q2.start()
        conv(1)
        own_r_q1 = copy(1, 1, my, 1, right)
        own_r_q1.start()
        conv(3)
        own_l_q3 = copy(7, 7, my, 3, left)
        own_l_q3.start()
        own_r_q2 = copy(2, 2, my, 2, right)
        own_r_q2.start()
        own_r_q3 = copy(3, 3, my, 3, right)
        own_r_q3.start()
        own_l_q0 = copy(4, 4, my, 0, left)
        own_l_q0.start()
        own_l_q1 = copy(5, 5, my, 1, left)
        own_l_q1.start()

        mm(my, 0, m_per)

        copy(0, 0, left, 0, left).wait_recv()
        fwd_r_q0 = copy(8, 8, left, 0, right)
        fwd_r_q0.start()
        copy(6, 6, right, 2, right).wait_recv()
        fwd_l_q2 = copy(10, 10, right, 2, left)
        fwd_l_q2.start()
        copy(1, 1, left, 1, left).wait_recv()
        fwd_r_q1 = copy(9, 9, left, 1, right)
        fwd_r_q1.start()
        copy(7, 7, right, 3, right).wait_recv()
        fwd_l_q3 = copy(11, 11, right, 3, left)
        fwd_l_q3.start()

        mm(left, 0, 2 * quart)
        mm(right, 2 * quart, 2 * quart)

        copy(2, 2, left, 2, left).wait_recv()
        copy(3, 3, left, 3, left).wait_recv()
        mm(left, 2 * quart, 2 * quart)
        copy(4, 4, right, 0, right).wait_recv()
        copy(5, 5, right, 1, right).wait_recv()
        mm(right, 0, 2 * quart)

        copy(8, 8, opp, 0, left).wait_recv()
        mm(opp, 0, quart)
        copy(10, 10, opp, 2, right).wait_recv()
        mm(opp, 2 * quart, quart)
        copy(9, 9, opp, 1, left).wait_recv()
        mm(opp, quart, quart)
        copy(11, 11, opp, 3, right).wait_recv()
        mm(opp, 3 * quart, quart)

        for s in (own_r_q0, own_r_q1, own_r_q2, own_r_q3,
                  own_l_q0, own_l_q1, own_l_q2, own_l_q3,
                  fwd_r_q0, fwd_r_q1, fwd_l_q2, fwd_l_q3):
            s.wait_send()

    return pl.pallas_call(
        body,
        out_shape=jax.ShapeDtypeStruct((M, n_per), jnp.float32),
        in_specs=[
            pl.BlockSpec(memory_space=pltpu.VMEM),
            pl.BlockSpec(memory_space=pltpu.VMEM),
        ],
        out_specs=pl.BlockSpec(memory_space=pltpu.VMEM),
        scratch_shapes=[
            pltpu.VMEM((M, k), jnp.bfloat16),
            pltpu.SemaphoreType.DMA((12,)),
            pltpu.SemaphoreType.DMA((12,)),
        ],
        compiler_params=pltpu.CompilerParams(
            collective_id=0,
            vmem_limit_bytes=63 * 1024 * 1024,
        ),
    )(x, w_mat)


# device time: 157404 ns/iter; 1.1159x vs baseline; 1.0472x over previous
import jax
import jax.numpy as jnp
from jax import lax
from jax.experimental import pallas as pl
from jax.experimental.pallas import tpu as pltpu

N_DEV = 4


def kernel(x, w_mat):
    m_per, k = x.shape
    _, n_per = w_mat.shape
    M = N_DEV * m_per
    quart = m_per // 4
    blk = 128
    wblk = k // 8

    def gelu(y):
        c = 0.7978845608028654
        return 0.5 * y * (1.0 + jnp.tanh(c * (y + 0.044715 * y * y * y)))

    def body(x_ref, w_ref, out_ref, xg_ref, wb_ref, stg_ref,
             send_sems, recv_sems, ldma_sems):
        my = lax.axis_index("i")
        left = lax.rem(my + N_DEV - 1, N_DEV)
        right = lax.rem(my + 1, N_DEV)
        opp = lax.rem(my + 2, N_DEV)

        barrier_sem = pltpu.get_barrier_semaphore()
        for nbr in (left, right):
            pl.semaphore_signal(
                barrier_sem, inc=1,
                device_id=(nbr,), device_id_type=pl.DeviceIdType.MESH,
            )

        def rows(o, start, size):
            return pl.ds(o * m_per + start, size)

        def copy(si, ri, origin, q, target):
            return pltpu.make_async_remote_copy(
                src_ref=xg_ref.at[rows(origin, q * quart, quart)],
                dst_ref=xg_ref.at[rows(origin, q * quart, quart)],
                send_sem=send_sems.at[si], recv_sem=recv_sems.at[ri],
                device_id=(target,), device_id_type=pl.DeviceIdType.MESH,
            )

        def mm(origin, start, size):
            acc = jnp.dot(
                xg_ref[rows(origin, start, size), :], wb_ref[...],
                preferred_element_type=jnp.float32,
            )
            out_ref[rows(origin, start, size), :] = gelu(acc)

        order = [0, 1, 4, 5, 2, 3, 6, 7]
        dmas = {}

        def start_dma(i):
            b = order[i]
            slot = i % 2
            d = pltpu.make_async_copy(
                x_ref.at[pl.ds(b * blk, blk)],
                stg_ref.at[slot],
                ldma_sems.at[slot],
            )
            d.start()
            dmas[i] = d

        start_dma(0)
        start_dma(1)

        own_sends = []
        for i in range(8):
            dmas[i].wait()
            b = order[i]
            xg_ref[rows(my, b * blk, blk), :] = (
                stg_ref[i % 2].astype(jnp.bfloat16)
            )
            if i + 2 < 8:
                start_dma(i + 2)
            if i == 1:
                pl.semaphore_wait(barrier_sem, 2)
                s = copy(0, 0, my, 0, right)
                s.start()
                own_sends.append(s)
            elif i == 3:
                s = copy(6, 6, my, 2, left)
                s.start()
                own_sends.append(s)
            elif i == 5:
                s = copy(1, 1, my, 1, right)
                s.start()
                own_sends.append(s)
            elif i == 7:
                for si, ri, q, tgt in (
                    (7, 7, 3, left),
                    (2, 2, 2, right),
                    (3, 3, 3, right),
                    (4, 4, 0, left),
                    (5, 5, 1, left),
                ):
                    s = copy(si, ri, my, q, tgt)
                    s.start()
                    own_sends.append(s)

        for j in range(k // wblk):
            wb_ref[pl.ds(j * wblk, wblk), :] = (
                w_ref[pl.ds(j * wblk, wblk), :].astype(jnp.bfloat16)
            )

        mm(my, 0, m_per)

        copy(0, 0, left, 0, left).wait_recv()
        fwd_r_q0 = copy(8, 8, left, 0, right)
        fwd_r_q0.start()
        copy(6, 6, right, 2, right).wait_recv()
        fwd_l_q2 = copy(10, 10, right, 2, left)
        fwd_l_q2.start()
        copy(1, 1, left, 1, left).wait_recv()
        fwd_r_q1 = copy(9, 9, left, 1, right)
        fwd_r_q1.start()
        copy(7, 7, right, 3, right).wait_recv()
        fwd_l_q3 = copy(11, 11, right, 3, left)
        fwd_l_q3.start()

        mm(left, 0, 2 * quart)
        mm(right, 2 * quart, 2 * quart)

        copy(2, 2, left, 2, left).wait_recv()
        copy(3, 3, left, 3, left).wait_recv()
        mm(left, 2 * quart, 2 * quart)
        copy(4, 4, right, 0, right).wait_recv()
        copy(5, 5, right, 1, right).wait_recv()
        mm(right, 0, 2 * quart)

        copy(8, 8, opp, 0, left).wait_recv()
        mm(opp, 0, quart)
        copy(10, 10, opp, 2, right).wait_recv()
        mm(opp, 2 * quart, quart)
        copy(9, 9, opp, 1, left).wait_recv()
        mm(opp, quart, quart)
        copy(11, 11, opp, 3, right).wait_recv()
        mm(opp, 3 * quart, quart)

        for s in own_sends + [fwd_r_q0, fwd_r_q1, fwd_l_q2, fwd_l_q3]:
            s.wait_send()

    return pl.pallas_call(
        body,
        out_shape=jax.ShapeDtypeStruct((M, n_per), jnp.float32),
        in_specs=[
            pl.BlockSpec(memory_space=pl.ANY),
            pl.BlockSpec(memory_space=pltpu.VMEM),
        ],
        out_specs=pl.BlockSpec(memory_space=pltpu.VMEM),
        scratch_shapes=[
            pltpu.VMEM((M, k), jnp.bfloat16),
            pltpu.VMEM((k, n_per), jnp.bfloat16),
            pltpu.VMEM((2, blk, k), jnp.float32),
            pltpu.SemaphoreType.DMA((12,)),
            pltpu.SemaphoreType.DMA((12,)),
            pltpu.SemaphoreType.DMA((2,)),
        ],
        compiler_params=pltpu.CompilerParams(
            collective_id=0,
            vmem_limit_bytes=63 * 1024 * 1024,
        ),
    )(x, w_mat)


# device time: 154000 ns/iter; 1.1406x vs baseline; 1.0221x over previous
import jax
import jax.numpy as jnp
from jax import lax
from jax.experimental import pallas as pl
from jax.experimental.pallas import tpu as pltpu

N_DEV = 4

PIECES = {
    0: (0, 128), 1: (128, 128), 2: (256, 256), 3: (512, 256), 4: (768, 256),
    5: (512, 128), 6: (640, 128), 7: (768, 256), 8: (0, 256), 9: (256, 256),
    10: (0, 256), 11: (256, 128), 12: (384, 128),
    13: (512, 256), 14: (768, 128), 15: (896, 128),
}


def kernel(x, w_mat):
    m_per, k = x.shape
    _, n_per = w_mat.shape
    M = N_DEV * m_per
    half = m_per // 2
    blk = 128
    wblk = k // 8

    def gelu(y):
        c = 0.7978845608028654
        return 0.5 * y * (1.0 + jnp.tanh(c * (y + 0.044715 * y * y * y)))

    def body(x_ref, w_ref, out_ref, xg_ref, wb_ref, stg_ref, outv_ref,
             send_sems, recv_sems, xdma_sems, odma_sems):
        my = lax.axis_index("i")
        left = lax.rem(my + N_DEV - 1, N_DEV)
        right = lax.rem(my + 1, N_DEV)
        opp = lax.rem(my + 2, N_DEV)

        barrier_sem = pltpu.get_barrier_semaphore()
        for nbr in (left, right):
            pl.semaphore_signal(
                barrier_sem, inc=1,
                device_id=(nbr,), device_id_type=pl.DeviceIdType.MESH,
            )

        def rows(o, start, size):
            return pl.ds(o * m_per + start, size)

        def copy(i, origin, target):
            start, size = PIECES[i]
            return pltpu.make_async_remote_copy(
                src_ref=xg_ref.at[rows(origin, start, size)],
                dst_ref=xg_ref.at[rows(origin, start, size)],
                send_sem=send_sems.at[i], recv_sem=recv_sems.at[i],
                device_id=(target,), device_id_type=pl.DeviceIdType.MESH,
            )

        out_dmas = []

        def mm(origin, start, size, oi):
            acc = jnp.dot(
                xg_ref[rows(origin, start, size), :], wb_ref[...],
                preferred_element_type=jnp.float32,
            )
            outv_ref[rows(origin, start, size), :] = gelu(acc)
            d = pltpu.make_async_copy(
                outv_ref.at[rows(origin, start, size)],
                out_ref.at[rows(origin, start, size)],
                odma_sems.at[oi],
            )
            d.start()
            out_dmas.append(d)

        order = [0, 1, 4, 5, 2, 3, 6, 7]
        dmas = {}

        def start_dma(i):
            b = order[i]
            slot = i % 2
            d = pltpu.make_async_copy(
                x_ref.at[pl.ds(b * blk, blk)],
                stg_ref.at[slot],
                xdma_sems.at[slot],
            )
            d.start()
            dmas[i] = d

        start_dma(0)
        start_dma(1)

        own_sends = []

        def send(i, target):
            s = copy(i, my, target)
            s.start()
            own_sends.append(s)

        for i in range(8):
            dmas[i].wait()
            b = order[i]
            xg_ref[rows(my, b * blk, blk), :] = (
                stg_ref[i % 2].astype(jnp.bfloat16)
            )
            if i + 2 < 8:
                start_dma(i + 2)
            if i == 0:
                pl.semaphore_wait(barrier_sem, 2)
                send(0, right)
            elif i == 1:
                send(1, right)
            elif i == 2:
                send(5, left)
            elif i == 3:
                send(6, left)
            elif i == 5:
                send(2, right)
            elif i == 7:
                send(7, left)
                send(3, right)
                send(4, right)
                send(8, left)
                send(9, left)

        for j in range(k // wblk):
            wb_ref[pl.ds(j * wblk, wblk), :] = (
                w_ref[pl.ds(j * wblk, wblk), :].astype(jnp.bfloat16)
            )

        mm(my, 0, m_per, 0)

        copy(0, left, left).wait_recv()
        copy(1, left, left).wait_recv()
        f1 = copy(10, left, right)
        f1.start()
        copy(5, right, right).wait_recv()
        copy(6, right, right).wait_recv()
        g1 = copy(13, right, left)
        g1.start()
        copy(2, left, left).wait_recv()
        f2 = copy(11, left, right)
        f2.start()
        f3 = copy(12, left, right)
        f3.start()
        copy(7, right, right).wait_recv()
        g2 = copy(14, right, left)
        g2.start()
        g3 = copy(15, right, left)
        g3.start()

        mm(left, 0, half, 1)
        mm(right, half, half, 2)

        copy(3, left, left).wait_recv()
        copy(4, left, left).wait_recv()
        mm(left, half, half, 3)
        copy(8, right, right).wait_recv()
        copy(9, right, right).wait_recv()
        mm(right, 0, half, 4)

        copy(10, opp, left).wait_recv()
        mm(opp, 0, 256, 5)
        copy(13, opp, right).wait_recv()
        mm(opp, 512, 256, 6)
        copy(11, opp, left).wait_recv()
        mm(opp, 256, 128, 7)
        copy(14, opp, right).wait_recv()
        mm(opp, 768, 128, 8)
        copy(12, opp, left).wait_recv()
        mm(opp, 384, 128, 9)
        copy(15, opp, right).wait_recv()
        mm(opp, 896, 128, 10)

        for s in own_sends + [f1, f2, f3, g1, g2, g3]:
            s.wait_send()
        for d in out_dmas:
            d.wait()

    return pl.pallas_call(
        body,
        out_shape=jax.ShapeDtypeStruct((M, n_per), jnp.float32),
        in_specs=[
            pl.BlockSpec(memory_space=pl.ANY),
            pl.BlockSpec(memory_space=pltpu.VMEM),
        ],
        out_specs=pl.BlockSpec(memory_space=pl.ANY),
        scratch_shapes=[
            pltpu.VMEM((M, k), jnp.bfloat16),
            pltpu.VMEM((k, n_per), jnp.bfloat16),
            pltpu.VMEM((2, blk, k), jnp.float32),
            pltpu.VMEM((M, n_per), jnp.float32),
            pltpu.SemaphoreType.DMA((16,)),
            pltpu.SemaphoreType.DMA((16,)),
            pltpu.SemaphoreType.DMA((2,)),
            pltpu.SemaphoreType.DMA((11,)),
        ],
        compiler_params=pltpu.CompilerParams(
            collective_id=0,
            vmem_limit_bytes=63 * 1024 * 1024,
        ),
    )(x, w_mat)
